# baseline (speedup 1.0000x reference)
import functools

import numpy as np
import jax
import jax.numpy as jnp
from jax import lax
from jax.experimental import pallas as pl
from jax.experimental.pallas import tpu as pltpu
from jax.experimental.pallas import tpu_sc as plsc

POS_VOCAB = 128


def _positions_np(n, vocab_size):
    lin = np.linspace(0.0, 1.0, n + 1, dtype=np.float32)
    intervals = np.stack([lin[:-1], lin[1:]]).T
    intervals = (intervals * vocab_size).astype(np.int32)
    intervals[:, 1] -= 1
    return np.round(intervals.astype(np.float32).mean(axis=-1)).astype(np.int32)


@functools.lru_cache(maxsize=None)
def _build_sc_gather(n_h, n_w, d, h_base, h_stride, w_base, w_stride):
    info = plsc.get_sparse_core_info()
    nc = info.num_cores
    mesh = plsc.VectorSubcoreMesh(core_axis_name="c", subcore_axis_name="s")

    @functools.partial(
        pl.kernel,
        mesh=mesh,
        out_type=(
            jax.ShapeDtypeStruct((n_h, 1, d), jnp.float32),
            jax.ShapeDtypeStruct((n_w, 1, d), jnp.float32),
        ),
    )
    def gather_kernel(htab, wtab, hsel, wsel):
        wid = lax.axis_index("s") * nc + lax.axis_index("c")

        @pl.when(wid == 0)
        def _():
            pltpu.sync_copy(htab.at[:, pl.ds(h_base, 1)], hsel)

        @pl.when(wid == 1)
        def _():
            pltpu.sync_copy(wtab.at[:, pl.ds(w_base, 1)], wsel)

    return gather_kernel


def _tc_add_body(hsel_ref, wsel_ref, out_ref):
    out_ref[...] = hsel_ref[...][:, None, :] + wsel_ref[...][None, :, :]


@functools.lru_cache(maxsize=None)
def _build_tc_add(n_h, n_w, d):
    return pl.pallas_call(
        _tc_add_body,
        out_shape=jax.ShapeDtypeStruct((n_h, n_w, d), jnp.float32),
    )


def kernel(x, height_table, width_table):
    n_h, n_w = x.shape[1], x.shape[2]
    d = height_table.shape[1]
    hpos = _positions_np(n_h, POS_VOCAB)
    wpos = _positions_np(n_w, POS_VOCAB)
    h_base, h_stride = int(hpos[0]), int(hpos[1] - hpos[0]) if n_h > 1 else 0
    w_base, w_stride = int(wpos[0]), int(wpos[1] - wpos[0]) if n_w > 1 else 0
    assert np.array_equal(hpos, h_base + h_stride * np.arange(n_h))
    assert np.array_equal(wpos, w_base + w_stride * np.arange(n_w))
    vocab = height_table.shape[0]
    assert h_stride == vocab // n_h and w_stride == vocab // n_w
    hsel, wsel = _build_sc_gather(n_h, n_w, d, h_base, h_stride, w_base, w_stride)(
        height_table.reshape(n_h, h_stride, d), width_table.reshape(n_w, w_stride, d)
    )
    return _build_tc_add(n_h, n_w, d)(hsel.reshape(n_h, d), wsel.reshape(n_w, d))


# speedup vs baseline: 1.0283x; 1.0283x over previous
import functools

import numpy as np
import jax
import jax.numpy as jnp
from jax import lax
from jax.experimental import pallas as pl
from jax.experimental.pallas import tpu as pltpu
from jax.experimental.pallas import tpu_sc as plsc

POS_VOCAB = 128


def _positions_np(n, vocab_size):
    lin = np.linspace(0.0, 1.0, n + 1, dtype=np.float32)
    intervals = np.stack([lin[:-1], lin[1:]]).T
    intervals = (intervals * vocab_size).astype(np.int32)
    intervals[:, 1] -= 1
    return np.round(intervals.astype(np.float32).mean(axis=-1)).astype(np.int32)


@functools.lru_cache(maxsize=None)
def _build_sc_gather(n_h, n_w, d, h_base, h_stride, w_base, w_stride):
    mesh = plsc.ScalarSubcoreMesh(axis_name="c")

    @functools.partial(
        pl.kernel,
        mesh=mesh,
        out_type=(
            jax.ShapeDtypeStruct((n_h, 1, d), jnp.float32),
            jax.ShapeDtypeStruct((n_w, 1, d), jnp.float32),
        ),
    )
    def gather_kernel(htab, wtab, hsel, wsel):
        cid = lax.axis_index("c")

        @pl.when(cid == 0)
        def _():
            pltpu.sync_copy(htab.at[:, pl.ds(h_base, 1)], hsel)

        @pl.when(cid == 1)
        def _():
            pltpu.sync_copy(wtab.at[:, pl.ds(w_base, 1)], wsel)

    return gather_kernel


def _tc_add_body(hsel_ref, wsel_ref, out_ref):
    out_ref[...] = hsel_ref[...][:, None, :] + wsel_ref[...][None, :, :]


@functools.lru_cache(maxsize=None)
def _build_tc_add(n_h, n_w, d):
    return pl.pallas_call(
        _tc_add_body,
        out_shape=jax.ShapeDtypeStruct((n_h, n_w, d), jnp.float32),
    )


def kernel(x, height_table, width_table):
    n_h, n_w = x.shape[1], x.shape[2]
    d = height_table.shape[1]
    hpos = _positions_np(n_h, POS_VOCAB)
    wpos = _positions_np(n_w, POS_VOCAB)
    h_base, h_stride = int(hpos[0]), int(hpos[1] - hpos[0]) if n_h > 1 else 0
    w_base, w_stride = int(wpos[0]), int(wpos[1] - wpos[0]) if n_w > 1 else 0
    assert np.array_equal(hpos, h_base + h_stride * np.arange(n_h))
    assert np.array_equal(wpos, w_base + w_stride * np.arange(n_w))
    vocab = height_table.shape[0]
    assert h_stride == vocab // n_h and w_stride == vocab // n_w
    hsel, wsel = _build_sc_gather(n_h, n_w, d, h_base, h_stride, w_base, w_stride)(
        height_table.reshape(n_h, h_stride, d), width_table.reshape(n_w, w_stride, d)
    )
    return _build_tc_add(n_h, n_w, d)(hsel.reshape(n_h, d), wsel.reshape(n_w, d))


# speedup vs baseline: 4.5787x; 4.4528x over previous
import functools

import numpy as np
import jax
import jax.numpy as jnp
from jax.experimental import pallas as pl

POS_VOCAB = 128


def _positions_np(n, vocab_size):
    lin = np.linspace(0.0, 1.0, n + 1, dtype=np.float32)
    intervals = np.stack([lin[:-1], lin[1:]]).T
    intervals = (intervals * vocab_size).astype(np.int32)
    intervals[:, 1] -= 1
    return np.round(intervals.astype(np.float32).mean(axis=-1)).astype(np.int32)


@functools.lru_cache(maxsize=None)
def _build(n_h, n_w, d, h_sub, w_sub):
    def body(htab_ref, wtab_ref, out_ref):
        hsel = htab_ref[:, h_sub, :]
        wsel = wtab_ref[:, w_sub, :]
        out_ref[...] = hsel[:, None, :] + wsel[None, :, :]

    return pl.pallas_call(
        body,
        out_shape=jax.ShapeDtypeStruct((n_h, n_w, d), jnp.float32),
    )


def kernel(x, height_table, width_table):
    n_h, n_w = x.shape[1], x.shape[2]
    d = height_table.shape[1]
    vocab = height_table.shape[0]
    hpos = _positions_np(n_h, POS_VOCAB)
    wpos = _positions_np(n_w, POS_VOCAB)
    h_base, h_stride = int(hpos[0]), int(hpos[1] - hpos[0]) if n_h > 1 else 0
    w_base, w_stride = int(wpos[0]), int(wpos[1] - wpos[0]) if n_w > 1 else 0
    assert np.array_equal(hpos, h_base + h_stride * np.arange(n_h))
    assert np.array_equal(wpos, w_base + w_stride * np.arange(n_w))
    assert h_stride == vocab // n_h and w_stride == vocab // n_w
    return _build(n_h, n_w, d, h_base, w_base)(
        height_table.reshape(n_h, h_stride, d), width_table.reshape(n_w, w_stride, d)
    )
